# Initial kernel scaffold; baseline (speedup 1.0000x reference)
import jax
import jax.numpy as jnp
from jax.experimental import pallas as pl


def kernel(heatmap, box_2d, boxes, labels):
    raise NotImplementedError("write your pallas kernel here")


# speedup vs baseline: 6.2096x; 6.2096x over previous
import jax
import jax.numpy as jnp
import numpy as np
from jax import lax
from jax.experimental import pallas as pl
from jax.experimental.pallas import tpu as pltpu

STRIDE = 4
NUM_CLASSES = 80
OUT_H = 128
OUT_W = 128
B = 8
N_BOXES = 32
R_MAX = 15

_DENOMS = np.asarray(
    [np.float32(2.0 * (r / 3 + 1 / 6) ** 2) for r in range(R_MAX + 1)], np.float32
)
_EPS = np.float32(np.finfo(np.float32).eps)
_NUMEL = float(B * NUM_CLASSES * OUT_H * OUT_W)

_I_SLOT, _I_RS, _I_CX, _I_CY, _I_RX, _I_RY, _I_RSY, _I_LAB = range(8)
_F_DENX, _F_DENY, _F_T0, _F_T1, _F_T2, _F_T3 = range(6)


def _body(ints_ref, flts_ref, hm_ref, b2_ref, out_ref, t_ref):
    b = pl.program_id(0)

    def _ssq_step(c, acc):
        x = hm_ref[0, c]
        return acc + jnp.sum(x * x)

    sumsq = lax.fori_loop(0, NUM_CLASSES, _ssq_step, jnp.float32(0.0))

    def _zero_step(j, _):
        t_ref[j] = jnp.zeros((OUT_H, OUT_W), jnp.float32)
        return 0

    lax.fori_loop(0, N_BOXES, _zero_step, 0)

    row_iota = lax.broadcasted_iota(jnp.int32, (40, OUT_W), 0)
    col_iota = lax.broadcasted_iota(jnp.int32, (40, OUT_W), 1)
    for i in range(N_BOXES):
        slot = ints_ref[b, i, _I_SLOT]
        rs = ints_ref[b, i, _I_RS]
        cx = ints_ref[b, i, _I_CX]
        cy = ints_ref[b, i, _I_CY]
        rx = ints_ref[b, i, _I_RX]
        ry = ints_ref[b, i, _I_RY]
        denx = flts_ref[b, i, _F_DENX]
        deny = flts_ref[b, i, _F_DENY]
        dy = (rs + row_iota) - cy
        dx = col_iota - cx
        e = dx.astype(jnp.float32) ** 2 / denx + dy.astype(jnp.float32) ** 2 / deny
        g = jnp.exp(-e)
        g = jnp.where(g < _EPS, jnp.float32(0.0), g)
        mask = (jnp.abs(dx) <= rx) & (jnp.abs(dy) <= ry)
        vals = jnp.where(mask, g, jnp.float32(0.0))
        cur = t_ref[slot, pl.ds(rs, 40), :]
        t_ref[slot, pl.ds(rs, 40), :] = jnp.maximum(cur, vals)

    def _corr_step(j, acc):
        lab = ints_ref[b, j, _I_LAB]
        tj = t_ref[j]
        hj = hm_ref[0, lab]
        return acc + jnp.sum(tj * (tj - 2.0 * hj))

    corr = lax.fori_loop(0, N_BOXES, _corr_step, jnp.float32(0.0))

    r16 = lax.broadcasted_iota(jnp.int32, (16, OUT_W), 0)
    c16 = lax.broadcasted_iota(jnp.int32, (16, OUT_W), 1)
    diff_sum = jnp.float32(0.0)
    cnt = jnp.float32(0.0)
    for i in range(N_BOXES):
        cx = ints_ref[b, i, _I_CX]
        cy = ints_ref[b, i, _I_CY]
        rsy = ints_ref[b, i, _I_RSY]
        t0 = flts_ref[b, i, _F_T0]
        t1 = flts_ref[b, i, _F_T1]
        t2 = flts_ref[b, i, _F_T2]
        t3 = flts_ref[b, i, _F_T3]
        ncy = rsy + r16
        m = (jnp.abs(ncy - cy) <= 1) & (jnp.abs(c16 - cx) <= 1)
        cxf = c16.astype(jnp.float32)
        cyf = ncy.astype(jnp.float32)
        strip = b2_ref[0, :, pl.ds(rsy, 16), :]
        s = jnp.float32(STRIDE)
        d = (
            jnp.abs((cxf - strip[0]) * s - t0)
            + jnp.abs((cyf - strip[1]) * s - t1)
            + jnp.abs((cxf + strip[2]) * s - t2)
            + jnp.abs((cyf + strip[3]) * s - t3)
        )
        mf = m.astype(jnp.float32)
        diff_sum = diff_sum + jnp.sum(d * mf)
        cnt = cnt + jnp.sum(mf)

    box_l = diff_sum / (cnt * jnp.float32(4.0))

    lane = lax.broadcasted_iota(jnp.int32, (1, 128), 1)
    row = (
        jnp.where(lane == 0, sumsq, jnp.float32(0.0))
        + jnp.where(lane == 1, corr, jnp.float32(0.0))
        + jnp.where(lane == 2, box_l, jnp.float32(0.0))
    )
    out_ref[0, 0] = row[0]


def kernel(heatmap, box_2d, boxes, labels):
    x = boxes[..., 0]
    y = boxes[..., 1]
    w = boxes[..., 2]
    h = boxes[..., 3]
    xs, ys, ws, hs = x / STRIDE, y / STRIDE, w / STRIDE, h / STRIDE
    cx = jnp.round(xs + ws / 2).astype(jnp.int32)
    cy = jnp.round(ys + hs / 2).astype(jnp.int32)
    rx = jnp.minimum(jnp.maximum(0, jnp.round(ws / 2 * 0.5).astype(jnp.int32)), R_MAX)
    ry = jnp.minimum(jnp.maximum(0, jnp.round(hs / 2 * 0.5).astype(jnp.int32)), R_MAX)
    table = jnp.asarray(_DENOMS)
    denx = table[rx]
    deny = table[ry]
    eq = labels[:, :, None] == labels[:, None, :]
    slot = jnp.argmax(eq, axis=-1).astype(jnp.int32)
    rs = jnp.clip(8 * ((cy - R_MAX) // 8), 0, OUT_H - 40).astype(jnp.int32)
    rsy = jnp.clip(8 * ((cy - 1) // 8), 0, OUT_H - 16).astype(jnp.int32)

    ints = jnp.stack([slot, rs, cx, cy, rx, ry, rsy, labels], axis=-1).astype(jnp.int32)
    flts = jnp.stack([denx, deny, x, y, x + w, y + h], axis=-1).astype(jnp.float32)

    out = pl.pallas_call(
        _body,
        grid=(B,),
        in_specs=[
            pl.BlockSpec(memory_space=pltpu.SMEM),
            pl.BlockSpec(memory_space=pltpu.SMEM),
            pl.BlockSpec(
                (1, NUM_CLASSES, OUT_H, OUT_W), lambda b: (b, 0, 0, 0)
            ),
            pl.BlockSpec((1, 4, OUT_H, OUT_W), lambda b: (b, 0, 0, 0)),
        ],
        out_specs=pl.BlockSpec((1, 1, 128), lambda b: (b, 0, 0)),
        out_shape=jax.ShapeDtypeStruct((B, 1, 128), jnp.float32),
        scratch_shapes=[pltpu.VMEM((N_BOXES, OUT_H, OUT_W), jnp.float32)],
        compiler_params=pltpu.CompilerParams(
            dimension_semantics=("arbitrary",),
        ),
    )(ints, flts, heatmap, box_2d)

    hm_loss = (jnp.sum(out[:, 0, 0]) + jnp.sum(out[:, 0, 1])) / jnp.float32(_NUMEL)
    box_loss = jnp.sum(out[:, 0, 2]) / jnp.float32(B)
    return jnp.stack([hm_loss, box_loss])


# speedup vs baseline: 23.0175x; 3.7068x over previous
import jax
import jax.numpy as jnp
import numpy as np
from jax import lax
from jax.experimental import pallas as pl
from jax.experimental.pallas import tpu as pltpu

STRIDE = 4
NUM_CLASSES = 80
OUT_H = 128
OUT_W = 128
B = 8
N_BOXES = 32
R_MAX = 15

_DENOMS = np.asarray(
    [np.float32(2.0 * (r / 3 + 1 / 6) ** 2) for r in range(R_MAX + 1)], np.float32
)
_EPS = np.float32(np.finfo(np.float32).eps)
_NUMEL = float(B * NUM_CLASSES * OUT_H * OUT_W)

_I_SLOT, _I_RS, _I_CX, _I_CY, _I_RX, _I_RY, _I_RSY, _I_LAB = range(8)
_F_DENX, _F_DENY, _F_T0, _F_T1, _F_T2, _F_T3 = range(6)


def _body(ints_ref, flts_ref, hm_ref, b2_ref, out_ref, t_ref):
    b = pl.program_id(0)

    def _ssq_step(c, acc):
        x = hm_ref[0, pl.ds(c * 8, 8)]
        return acc + jnp.sum(x * x, axis=0)

    ssq_vec = lax.fori_loop(
        0, NUM_CLASSES // 8, _ssq_step, jnp.zeros((OUT_H, OUT_W), jnp.float32)
    )
    sumsq = jnp.sum(ssq_vec)

    def _zero_step(j, _):
        t_ref[j] = jnp.zeros((OUT_H, OUT_W), jnp.float32)
        return 0

    lax.fori_loop(0, N_BOXES, _zero_step, 0)

    row_iota = lax.broadcasted_iota(jnp.int32, (40, OUT_W), 0)
    col_iota = lax.broadcasted_iota(jnp.int32, (40, OUT_W), 1)
    for i in range(N_BOXES):
        slot = ints_ref[b, i, _I_SLOT]
        rs = ints_ref[b, i, _I_RS]
        cx = ints_ref[b, i, _I_CX]
        cy = ints_ref[b, i, _I_CY]
        rx = ints_ref[b, i, _I_RX]
        ry = ints_ref[b, i, _I_RY]
        denx = flts_ref[b, i, _F_DENX]
        deny = flts_ref[b, i, _F_DENY]
        dy = (rs + row_iota) - cy
        dx = col_iota - cx
        e = dx.astype(jnp.float32) ** 2 / denx + dy.astype(jnp.float32) ** 2 / deny
        g = jnp.exp(-e)
        g = jnp.where(g < _EPS, jnp.float32(0.0), g)
        mask = (jnp.abs(dx) <= rx) & (jnp.abs(dy) <= ry)
        vals = jnp.where(mask, g, jnp.float32(0.0))
        cur = t_ref[slot, pl.ds(rs, 40), :]
        t_ref[slot, pl.ds(rs, 40), :] = jnp.maximum(cur, vals)

    def _corr_step(j, acc):
        lab = ints_ref[b, j, _I_LAB]
        tj = t_ref[j]
        hj = hm_ref[0, lab]
        return acc + tj * (tj - 2.0 * hj)

    corr_vec = lax.fori_loop(
        0, N_BOXES, _corr_step, jnp.zeros((OUT_H, OUT_W), jnp.float32)
    )
    corr = jnp.sum(corr_vec)

    r16 = lax.broadcasted_iota(jnp.int32, (16, OUT_W), 0)
    c16 = lax.broadcasted_iota(jnp.int32, (16, OUT_W), 1)
    diff_acc = jnp.zeros((16, OUT_W), jnp.float32)
    cnt_acc = jnp.zeros((16, OUT_W), jnp.float32)
    for i in range(N_BOXES):
        cx = ints_ref[b, i, _I_CX]
        cy = ints_ref[b, i, _I_CY]
        rsy = ints_ref[b, i, _I_RSY]
        t0 = flts_ref[b, i, _F_T0]
        t1 = flts_ref[b, i, _F_T1]
        t2 = flts_ref[b, i, _F_T2]
        t3 = flts_ref[b, i, _F_T3]
        ncy = rsy + r16
        m = (jnp.abs(ncy - cy) <= 1) & (jnp.abs(c16 - cx) <= 1)
        cxf = c16.astype(jnp.float32)
        cyf = ncy.astype(jnp.float32)
        strip = b2_ref[0, :, pl.ds(rsy, 16), :]
        s = jnp.float32(STRIDE)
        d = (
            jnp.abs((cxf - strip[0]) * s - t0)
            + jnp.abs((cyf - strip[1]) * s - t1)
            + jnp.abs((cxf + strip[2]) * s - t2)
            + jnp.abs((cyf + strip[3]) * s - t3)
        )
        mf = m.astype(jnp.float32)
        diff_acc = diff_acc + d * mf
        cnt_acc = cnt_acc + mf

    box_l = jnp.sum(diff_acc) / (jnp.sum(cnt_acc) * jnp.float32(4.0))

    lane = lax.broadcasted_iota(jnp.int32, (1, 128), 1)
    row = (
        jnp.where(lane == 0, sumsq, jnp.float32(0.0))
        + jnp.where(lane == 1, corr, jnp.float32(0.0))
        + jnp.where(lane == 2, box_l, jnp.float32(0.0))
    )
    out_ref[0, 0] = row[0]


def kernel(heatmap, box_2d, boxes, labels):
    x = boxes[..., 0]
    y = boxes[..., 1]
    w = boxes[..., 2]
    h = boxes[..., 3]
    xs, ys, ws, hs = x / STRIDE, y / STRIDE, w / STRIDE, h / STRIDE
    cx = jnp.round(xs + ws / 2).astype(jnp.int32)
    cy = jnp.round(ys + hs / 2).astype(jnp.int32)
    rx = jnp.minimum(jnp.maximum(0, jnp.round(ws / 2 * 0.5).astype(jnp.int32)), R_MAX)
    ry = jnp.minimum(jnp.maximum(0, jnp.round(hs / 2 * 0.5).astype(jnp.int32)), R_MAX)
    table = jnp.asarray(_DENOMS)
    denx = table[rx]
    deny = table[ry]
    eq = labels[:, :, None] == labels[:, None, :]
    slot = jnp.argmax(eq, axis=-1).astype(jnp.int32)
    rs = jnp.clip(8 * ((cy - R_MAX) // 8), 0, OUT_H - 40).astype(jnp.int32)
    rsy = jnp.clip(8 * ((cy - 1) // 8), 0, OUT_H - 16).astype(jnp.int32)

    ints = jnp.stack([slot, rs, cx, cy, rx, ry, rsy, labels], axis=-1).astype(jnp.int32)
    flts = jnp.stack([denx, deny, x, y, x + w, y + h], axis=-1).astype(jnp.float32)

    out = pl.pallas_call(
        _body,
        grid=(B,),
        in_specs=[
            pl.BlockSpec(memory_space=pltpu.SMEM),
            pl.BlockSpec(memory_space=pltpu.SMEM),
            pl.BlockSpec(
                (1, NUM_CLASSES, OUT_H, OUT_W), lambda b: (b, 0, 0, 0)
            ),
            pl.BlockSpec((1, 4, OUT_H, OUT_W), lambda b: (b, 0, 0, 0)),
        ],
        out_specs=pl.BlockSpec((1, 1, 128), lambda b: (b, 0, 0)),
        out_shape=jax.ShapeDtypeStruct((B, 1, 128), jnp.float32),
        scratch_shapes=[pltpu.VMEM((N_BOXES, OUT_H, OUT_W), jnp.float32)],
        compiler_params=pltpu.CompilerParams(
            dimension_semantics=("arbitrary",),
        ),
    )(ints, flts, heatmap, box_2d)

    hm_loss = (jnp.sum(out[:, 0, 0]) + jnp.sum(out[:, 0, 1])) / jnp.float32(_NUMEL)
    box_loss = jnp.sum(out[:, 0, 2]) / jnp.float32(B)
    return jnp.stack([hm_loss, box_loss])


# speedup vs baseline: 23.0366x; 1.0008x over previous
import jax
import jax.numpy as jnp
import numpy as np
from jax import lax
from jax.experimental import pallas as pl
from jax.experimental.pallas import tpu as pltpu

STRIDE = 4
NUM_CLASSES = 80
OUT_H = 128
OUT_W = 128
B = 8
N_BOXES = 32
R_MAX = 15

_DENOMS = np.asarray(
    [np.float32(2.0 * (r / 3 + 1 / 6) ** 2) for r in range(R_MAX + 1)], np.float32
)
_EPS = np.float32(np.finfo(np.float32).eps)
_NUMEL = float(B * NUM_CLASSES * OUT_H * OUT_W)

_I_SLOT, _I_RS, _I_CX, _I_CY, _I_RX, _I_RY, _I_RSY, _I_LAB = range(8)
_F_DENX, _F_DENY, _F_T0, _F_T1, _F_T2, _F_T3 = range(6)


def _body(ints_ref, flts_ref, hm_ref, b2_ref, out_ref, t_ref):
    b = pl.program_id(0)

    def _ssq_step(c, acc):
        x = hm_ref[0, pl.ds(c * 8, 8)]
        return acc + jnp.sum(x * x, axis=0)

    ssq_vec = lax.fori_loop(
        0, NUM_CLASSES // 8, _ssq_step, jnp.zeros((OUT_H, OUT_W), jnp.float32)
    )
    sumsq = jnp.sum(ssq_vec)

    def _zero_step(j, _):
        t_ref[j] = jnp.zeros((OUT_H, OUT_W), jnp.float32)
        return 0

    lax.fori_loop(0, N_BOXES, _zero_step, 0)

    row_iota = lax.broadcasted_iota(jnp.int32, (40, OUT_W), 0)
    col_iota = lax.broadcasted_iota(jnp.int32, (40, OUT_W), 1)
    for i in range(N_BOXES):
        slot = ints_ref[b, i, _I_SLOT]
        rs = ints_ref[b, i, _I_RS]
        cx = ints_ref[b, i, _I_CX]
        cy = ints_ref[b, i, _I_CY]
        rx = ints_ref[b, i, _I_RX]
        ry = ints_ref[b, i, _I_RY]
        denx = flts_ref[b, i, _F_DENX]
        deny = flts_ref[b, i, _F_DENY]
        dy = (rs + row_iota) - cy
        dx = col_iota - cx
        e = dx.astype(jnp.float32) ** 2 / denx + dy.astype(jnp.float32) ** 2 / deny
        g = jnp.exp(-e)
        g = jnp.where(g < _EPS, jnp.float32(0.0), g)
        mask = (jnp.abs(dx) <= rx) & (jnp.abs(dy) <= ry)
        vals = jnp.where(mask, g, jnp.float32(0.0))
        cur = t_ref[slot, pl.ds(rs, 40), :]
        t_ref[slot, pl.ds(rs, 40), :] = jnp.maximum(cur, vals)

    def _corr_step(j, acc):
        lab = ints_ref[b, j, _I_LAB]
        tj = t_ref[j]
        hj = hm_ref[0, lab]
        return acc + tj * (tj - 2.0 * hj)

    corr_vec = lax.fori_loop(
        0, N_BOXES, _corr_step, jnp.zeros((OUT_H, OUT_W), jnp.float32)
    )
    corr = jnp.sum(corr_vec)

    r16 = lax.broadcasted_iota(jnp.int32, (16, OUT_W), 0)
    c16 = lax.broadcasted_iota(jnp.int32, (16, OUT_W), 1)
    diff_acc = jnp.zeros((16, OUT_W), jnp.float32)
    cnt_acc = jnp.zeros((16, OUT_W), jnp.float32)
    for i in range(N_BOXES):
        cx = ints_ref[b, i, _I_CX]
        cy = ints_ref[b, i, _I_CY]
        rsy = ints_ref[b, i, _I_RSY]
        t0 = flts_ref[b, i, _F_T0]
        t1 = flts_ref[b, i, _F_T1]
        t2 = flts_ref[b, i, _F_T2]
        t3 = flts_ref[b, i, _F_T3]
        ncy = rsy + r16
        m = (jnp.abs(ncy - cy) <= 1) & (jnp.abs(c16 - cx) <= 1)
        cxf = c16.astype(jnp.float32)
        cyf = ncy.astype(jnp.float32)
        strip = b2_ref[0, :, pl.ds(rsy, 16), :]
        s = jnp.float32(STRIDE)
        d = (
            jnp.abs((cxf - strip[0]) * s - t0)
            + jnp.abs((cyf - strip[1]) * s - t1)
            + jnp.abs((cxf + strip[2]) * s - t2)
            + jnp.abs((cyf + strip[3]) * s - t3)
        )
        mf = m.astype(jnp.float32)
        diff_acc = diff_acc + d * mf
        cnt_acc = cnt_acc + mf

    box_l = jnp.sum(diff_acc) / (jnp.sum(cnt_acc) * jnp.float32(4.0))

    lane = lax.broadcasted_iota(jnp.int32, (1, 128), 1)
    row = (
        jnp.where(lane == 0, sumsq, jnp.float32(0.0))
        + jnp.where(lane == 1, corr, jnp.float32(0.0))
        + jnp.where(lane == 2, box_l, jnp.float32(0.0))
    )
    out_ref[0, 0] = row[0]


def kernel(heatmap, box_2d, boxes, labels):
    x = boxes[..., 0]
    y = boxes[..., 1]
    w = boxes[..., 2]
    h = boxes[..., 3]
    xs, ys, ws, hs = x / STRIDE, y / STRIDE, w / STRIDE, h / STRIDE
    cx = jnp.round(xs + ws / 2).astype(jnp.int32)
    cy = jnp.round(ys + hs / 2).astype(jnp.int32)
    rx = jnp.minimum(jnp.maximum(0, jnp.round(ws / 2 * 0.5).astype(jnp.int32)), R_MAX)
    ry = jnp.minimum(jnp.maximum(0, jnp.round(hs / 2 * 0.5).astype(jnp.int32)), R_MAX)
    table = jnp.asarray(_DENOMS)
    denx = table[rx]
    deny = table[ry]
    eq = labels[:, :, None] == labels[:, None, :]
    slot = jnp.argmax(eq, axis=-1).astype(jnp.int32)
    rs = jnp.clip(8 * ((cy - R_MAX) // 8), 0, OUT_H - 40).astype(jnp.int32)
    rsy = jnp.clip(8 * ((cy - 1) // 8), 0, OUT_H - 16).astype(jnp.int32)

    ints = jnp.stack([slot, rs, cx, cy, rx, ry, rsy, labels], axis=-1).astype(jnp.int32)
    flts = jnp.stack([denx, deny, x, y, x + w, y + h], axis=-1).astype(jnp.float32)

    out = pl.pallas_call(
        _body,
        grid=(B,),
        in_specs=[
            pl.BlockSpec(memory_space=pltpu.SMEM),
            pl.BlockSpec(memory_space=pltpu.SMEM),
            pl.BlockSpec(
                (1, NUM_CLASSES, OUT_H, OUT_W), lambda b: (b, 0, 0, 0)
            ),
            pl.BlockSpec((1, 4, OUT_H, OUT_W), lambda b: (b, 0, 0, 0)),
        ],
        out_specs=pl.BlockSpec((1, 1, 128), lambda b: (b, 0, 0)),
        out_shape=jax.ShapeDtypeStruct((B, 1, 128), jnp.float32),
        scratch_shapes=[pltpu.VMEM((N_BOXES, OUT_H, OUT_W), jnp.float32)],
        compiler_params=pltpu.CompilerParams(
            dimension_semantics=("parallel",),
        ),
    )(ints, flts, heatmap, box_2d)

    hm_loss = (jnp.sum(out[:, 0, 0]) + jnp.sum(out[:, 0, 1])) / jnp.float32(_NUMEL)
    box_loss = jnp.sum(out[:, 0, 2]) / jnp.float32(B)
    return jnp.stack([hm_loss, box_loss])
